# baseline (speedup 1.0000x reference)
import functools

import jax
import jax.numpy as jnp
from jax import lax
from jax.experimental import pallas as pl
from jax.experimental.pallas import tpu as pltpu
from jax.experimental.pallas import tpu_sc as plsc

T = 2048
D = 768
FF = 1024
E = 64
K = 2
TK = T * K
TM = 256
NT = TK // TM
NS = E + NT - 1
NMETA = 128

_SQRT_HALF = 0.7071067811865476


def _router_body(hs_ref, gw_ref, w0_ref, w1_ref, d0_ref, d1_ref, meta_ref):
    x = hs_ref[...]
    logits = lax.dot_general(x, gw_ref[...], (((1,), (1,)), ((), ())),
                             preferred_element_type=jnp.float32)
    col = lax.broadcasted_iota(jnp.int32, (T, E), 1)
    m1 = jnp.max(logits, axis=1, keepdims=True)
    a1 = jnp.min(jnp.where(logits == m1, col, E), axis=1, keepdims=True)
    l2 = jnp.where(col == a1, -jnp.inf, logits)
    m2 = jnp.max(l2, axis=1, keepdims=True)
    a2 = jnp.min(jnp.where(l2 == m2, col, E), axis=1, keepdims=True)
    w1 = 1.0 / (1.0 + jnp.exp(m2 - m1))
    w0_ref[...] = jnp.broadcast_to(w1, (T, 16))
    w1_ref[...] = jnp.broadcast_to(1.0 - w1, (T, 16))

    oh1 = (col == a1).astype(jnp.float32)
    oh2 = (col == a2).astype(jnp.float32)
    oh = oh1 + oh2

    cum = oh
    sh = 1
    while sh < T:
        cum = cum + jnp.concatenate(
            [jnp.zeros((sh, E), jnp.float32), cum[: T - sh]], axis=0)
        sh *= 2
    cumex = cum - oh
    counts = cum[T - 1:T, :]

    lane = lax.broadcasted_iota(jnp.int32, (1, E), 1)
    cseg = counts
    sh = 1
    while sh < E:
        cseg = cseg + jnp.concatenate(
            [jnp.zeros((1, sh), jnp.float32), cseg[:, : E - sh]], axis=1)
        sh *= 2
    ends = cseg
    starts = ends - counts

    st1 = jnp.sum(oh1 * starts, axis=1, keepdims=True)
    st2 = jnp.sum(oh2 * starts, axis=1, keepdims=True)
    ce1 = jnp.sum(oh1 * cumex, axis=1, keepdims=True)
    ce2 = jnp.sum(oh2 * cumex, axis=1, keepdims=True)
    d0_ref[...] = (st1 + ce1).astype(jnp.int32)
    d1_ref[...] = (st2 + ce2).astype(jnp.int32)

    inv_tm = 1.0 / TM
    first_tile = jnp.floor(starts * inv_tm)
    last_tile = jnp.floor(jnp.maximum(ends - 1.0, 0.0) * inv_tm)
    tiles_e = jnp.where(counts > 0.0, last_tile - first_tile + 1.0, 0.0)
    ctile = tiles_e
    sh = 1
    while sh < E:
        ctile = ctile + jnp.concatenate(
            [jnp.zeros((1, sh), jnp.float32), ctile[:, : E - sh]], axis=1)
        sh *= 2
    step_base = ctile - tiles_e
    total = jnp.sum(tiles_e, axis=1, keepdims=True)

    s_col = lax.broadcasted_iota(jnp.int32, (NMETA, 1), 0).astype(jnp.float32)
    s_c = jnp.minimum(s_col, total - 1.0)
    e_of_s = jnp.sum((ctile <= s_c).astype(jnp.float32), axis=1, keepdims=True)
    e_of_s = jnp.clip(e_of_s, 0.0, E - 1.0)
    ohs = (lax.broadcasted_iota(jnp.int32, (NMETA, E), 1)
           == e_of_s.astype(jnp.int32)).astype(jnp.float32)
    ft_g = jnp.sum(ohs * first_tile, axis=1, keepdims=True)
    sb_g = jnp.sum(ohs * step_base, axis=1, keepdims=True)
    st_g = jnp.sum(ohs * starts, axis=1, keepdims=True)
    en_g = jnp.sum(ohs * ends, axis=1, keepdims=True)
    tile_of_s = ft_g + s_c - sb_g
    t0 = tile_of_s * float(TM)
    lo = jnp.clip(st_g - t0, 0.0, float(TM))
    hi = jnp.clip(en_g - t0, 0.0, float(TM))
    hi = jnp.where(s_col < total, hi, lo)
    pad = jnp.zeros((NMETA, 4), jnp.float32)
    meta_ref[...] = jnp.concatenate(
        [e_of_s, tile_of_s, lo, hi, pad], axis=1).astype(jnp.int32)


def _router(hs2d, gate_w):
    return pl.pallas_call(
        _router_body,
        out_shape=(jax.ShapeDtypeStruct((T, 16), jnp.float32),
                   jax.ShapeDtypeStruct((T, 16), jnp.float32),
                   jax.ShapeDtypeStruct((T, 1), jnp.int32),
                   jax.ShapeDtypeStruct((T, 1), jnp.int32),
                   jax.ShapeDtypeStruct((NMETA, 8), jnp.int32)),
    )(hs2d, gate_w)


def _ffn_body(meta_ref, xs_ref, wg_ref, wu_ref, wd_ref, out_ref):
    s = pl.program_id(0)
    lo = meta_ref[s, 2]
    hi = meta_ref[s, 3]
    x = xs_ref[...].astype(jnp.bfloat16)
    wg = wg_ref[0].astype(jnp.bfloat16)
    wu = wu_ref[0].astype(jnp.bfloat16)
    wd = wd_ref[0].astype(jnp.bfloat16)
    h = lax.dot_general(x, wg, (((1,), (1,)), ((), ())),
                        preferred_element_type=jnp.float32)
    u = lax.dot_general(x, wu, (((1,), (1,)), ((), ())),
                        preferred_element_type=jnp.float32)
    g = 0.5 * h * (1.0 + lax.erf(h * _SQRT_HALF))
    y = lax.dot_general((g * u).astype(jnp.bfloat16), wd,
                        (((1,), (1,)), ((), ())),
                        preferred_element_type=jnp.float32)
    row = lax.broadcasted_iota(jnp.int32, (TM, 1), 0)
    mask = (row >= lo) & (row < hi)
    out_ref[...] = jnp.where(mask, y, out_ref[...])


def _grouped_ffn(meta, xs, Wg, Wu, Wd):
    grid_spec = pltpu.PrefetchScalarGridSpec(
        num_scalar_prefetch=1,
        grid=(NS,),
        in_specs=[
            pl.BlockSpec((TM, D), lambda s, m: (m[s, 1], 0)),
            pl.BlockSpec((1, FF, D), lambda s, m: (m[s, 0], 0, 0)),
            pl.BlockSpec((1, FF, D), lambda s, m: (m[s, 0], 0, 0)),
            pl.BlockSpec((1, D, FF), lambda s, m: (m[s, 0], 0, 0)),
        ],
        out_specs=pl.BlockSpec((TM, D), lambda s, m: (m[s, 1], 0)),
    )
    return pl.pallas_call(
        _ffn_body,
        grid_spec=grid_spec,
        out_shape=jax.ShapeDtypeStruct((TK, D), jnp.float32),
        compiler_params=pltpu.CompilerParams(
            dimension_semantics=("arbitrary",)),
    )(meta, xs, Wg, Wu, Wd)


_NW = 32
_CPW = T // _NW


@functools.cache
def _sc_kernels():
    mesh = plsc.VectorSubcoreMesh(core_axis_name="c", subcore_axis_name="s")

    @functools.partial(
        pl.kernel, mesh=mesh,
        out_type=jax.ShapeDtypeStruct((TK, D), jnp.float32),
        scratch_types=[
            pltpu.VMEM((_CPW,), jnp.int32),
            pltpu.VMEM((_CPW,), jnp.int32),
            pltpu.VMEM((_CPW, D), jnp.float32),
            pltpu.SemaphoreType.DMA,
            pltpu.SemaphoreType.DMA,
            pltpu.SemaphoreType.DMA,
        ],
    )
    def sc_scatter(hs_hbm, d0_hbm, d1_hbm, out_hbm,
                   i0_v, i1_v, rows_v, s0, s1, s2):
        wid = lax.axis_index("s") * 2 + lax.axis_index("c")
        base = wid * _CPW
        a0 = pltpu.async_copy(d0_hbm.at[pl.ds(base, _CPW)], i0_v, s0)
        a1 = pltpu.async_copy(d1_hbm.at[pl.ds(base, _CPW)], i1_v, s1)
        a2 = pltpu.async_copy(hs_hbm.at[pl.ds(base, _CPW)], rows_v, s2)
        a0.wait()
        a1.wait()
        a2.wait()
        c0 = pltpu.async_copy(rows_v, out_hbm.at[i0_v], s0)
        c1 = pltpu.async_copy(rows_v, out_hbm.at[i1_v], s1)
        c0.wait()
        c1.wait()

    @functools.partial(
        pl.kernel, mesh=mesh,
        out_type=jax.ShapeDtypeStruct((T, D), jnp.float32),
        scratch_types=[
            pltpu.VMEM((_CPW,), jnp.int32),
            pltpu.VMEM((_CPW,), jnp.int32),
            pltpu.VMEM((_CPW, 16), jnp.float32),
            pltpu.VMEM((_CPW, 16), jnp.float32),
            pltpu.VMEM((_CPW, D), jnp.float32),
            pltpu.VMEM((_CPW, D), jnp.float32),
            pltpu.SemaphoreType.DMA,
            pltpu.SemaphoreType.DMA,
        ],
    )
    def sc_combine(ysw_hbm, pos0_hbm, pos1_hbm, w0_hbm, w1_hbm, out_hbm,
                   i0_v, i1_v, w0_v, w1_v, r0_v, r1_v, s0, s1):
        wid = lax.axis_index("s") * 2 + lax.axis_index("c")
        base = wid * _CPW
        a0 = pltpu.async_copy(pos0_hbm.at[pl.ds(base, _CPW)], i0_v, s0)
        a1 = pltpu.async_copy(pos1_hbm.at[pl.ds(base, _CPW)], i1_v, s1)
        pltpu.sync_copy(w0_hbm.at[pl.ds(base, _CPW)], w0_v)
        pltpu.sync_copy(w1_hbm.at[pl.ds(base, _CPW)], w1_v)
        a0.wait()
        a1.wait()
        c0 = pltpu.async_copy(ysw_hbm.at[i0_v], r0_v, s0)
        c1 = pltpu.async_copy(ysw_hbm.at[i1_v], r1_v, s1)
        c0.wait()
        c1.wait()

        def row_add(j, carry):
            a = w0_v[j, pl.ds(0, 16)]
            b = w1_v[j, pl.ds(0, 16)]
            for c in range(D // 16):
                sl = pl.ds(c * 16, 16)
                r0_v[j, sl] = r0_v[j, sl] * a + r1_v[j, sl] * b
            return carry

        lax.fori_loop(0, _CPW, row_add, 0)
        pltpu.sync_copy(r0_v, out_hbm.at[pl.ds(base, _CPW)])

    return sc_scatter, sc_combine


def kernel(hidden_states, gate_w, Wg, Wu, Wd):
    b, s, d = hidden_states.shape
    hs2d = hidden_states.reshape(T, D)
    w0, w1, dest0, dest1, meta = _router(hs2d, gate_w)
    d0 = dest0.reshape(T)
    d1 = dest1.reshape(T)
    sc_scatter, sc_combine = _sc_kernels()
    xs = sc_scatter(hs2d, d0, d1)
    ysw = _grouped_ffn(meta, xs, Wg, Wu, Wd)
    final = sc_combine(ysw, d0, d1, w0, w1)
    return final.reshape(b, s, d)


# speedup vs baseline: 1.0076x; 1.0076x over previous
import functools

import jax
import jax.numpy as jnp
from jax import lax
from jax.experimental import pallas as pl
from jax.experimental.pallas import tpu as pltpu
from jax.experimental.pallas import tpu_sc as plsc

T = 2048
D = 768
FF = 1024
E = 64
K = 2
TK = T * K
TM = 256
NT = TK // TM
NS = E + NT - 1
NMETA = 128

_SQRT_HALF = 0.7071067811865476


def _router_body(hs_ref, gw_ref, w0_ref, w1_ref, d0_ref, d1_ref, meta_ref):
    x = hs_ref[...]
    logits = lax.dot_general(x, gw_ref[...], (((1,), (1,)), ((), ())),
                             preferred_element_type=jnp.float32)
    col = lax.broadcasted_iota(jnp.int32, (T, E), 1)
    m1 = jnp.max(logits, axis=1, keepdims=True)
    a1 = jnp.min(jnp.where(logits == m1, col, E), axis=1, keepdims=True)
    l2 = jnp.where(col == a1, -jnp.inf, logits)
    m2 = jnp.max(l2, axis=1, keepdims=True)
    a2 = jnp.min(jnp.where(l2 == m2, col, E), axis=1, keepdims=True)
    w1 = 1.0 / (1.0 + jnp.exp(m2 - m1))
    w0_ref[...] = jnp.broadcast_to(w1, (T, 16))
    w1_ref[...] = jnp.broadcast_to(1.0 - w1, (T, 16))

    oh1 = (col == a1).astype(jnp.float32)
    oh2 = (col == a2).astype(jnp.float32)
    oh = oh1 + oh2

    cum = oh
    sh = 1
    while sh < T:
        cum = cum + jnp.concatenate(
            [jnp.zeros((sh, E), jnp.float32), cum[: T - sh]], axis=0)
        sh *= 2
    cumex = cum - oh
    counts = cum[T - 1:T, :]

    lane = lax.broadcasted_iota(jnp.int32, (1, E), 1)
    cseg = counts
    sh = 1
    while sh < E:
        cseg = cseg + jnp.concatenate(
            [jnp.zeros((1, sh), jnp.float32), cseg[:, : E - sh]], axis=1)
        sh *= 2
    ends = cseg
    starts = ends - counts

    st1 = jnp.sum(oh1 * starts, axis=1, keepdims=True)
    st2 = jnp.sum(oh2 * starts, axis=1, keepdims=True)
    ce1 = jnp.sum(oh1 * cumex, axis=1, keepdims=True)
    ce2 = jnp.sum(oh2 * cumex, axis=1, keepdims=True)
    d0_ref[...] = (st1 + ce1).astype(jnp.int32)
    d1_ref[...] = (st2 + ce2).astype(jnp.int32)

    inv_tm = 1.0 / TM
    first_tile = jnp.floor(starts * inv_tm)
    last_tile = jnp.floor(jnp.maximum(ends - 1.0, 0.0) * inv_tm)
    tiles_e = jnp.where(counts > 0.0, last_tile - first_tile + 1.0, 0.0)
    ctile = tiles_e
    sh = 1
    while sh < E:
        ctile = ctile + jnp.concatenate(
            [jnp.zeros((1, sh), jnp.float32), ctile[:, : E - sh]], axis=1)
        sh *= 2
    step_base = ctile - tiles_e
    total = jnp.sum(tiles_e, axis=1, keepdims=True)

    s_col = lax.broadcasted_iota(jnp.int32, (NMETA, 1), 0).astype(jnp.float32)
    s_c = jnp.minimum(s_col, total - 1.0)
    e_of_s = jnp.sum((ctile <= s_c).astype(jnp.float32), axis=1, keepdims=True)
    e_of_s = jnp.clip(e_of_s, 0.0, E - 1.0)
    ohs = (lax.broadcasted_iota(jnp.int32, (NMETA, E), 1)
           == e_of_s.astype(jnp.int32)).astype(jnp.float32)
    ft_g = jnp.sum(ohs * first_tile, axis=1, keepdims=True)
    sb_g = jnp.sum(ohs * step_base, axis=1, keepdims=True)
    st_g = jnp.sum(ohs * starts, axis=1, keepdims=True)
    en_g = jnp.sum(ohs * ends, axis=1, keepdims=True)
    tile_of_s = ft_g + s_c - sb_g
    t0 = tile_of_s * float(TM)
    lo = jnp.clip(st_g - t0, 0.0, float(TM))
    hi = jnp.clip(en_g - t0, 0.0, float(TM))
    hi = jnp.where(s_col < total, hi, lo)
    pad = jnp.zeros((NMETA, 4), jnp.float32)
    meta_ref[...] = jnp.concatenate(
        [e_of_s, tile_of_s, lo, hi, pad], axis=1).astype(jnp.int32)


def _router(hs2d, gate_w):
    return pl.pallas_call(
        _router_body,
        out_shape=(jax.ShapeDtypeStruct((T, 16), jnp.float32),
                   jax.ShapeDtypeStruct((T, 16), jnp.float32),
                   jax.ShapeDtypeStruct((T, 1), jnp.int32),
                   jax.ShapeDtypeStruct((T, 1), jnp.int32),
                   jax.ShapeDtypeStruct((NMETA, 8), jnp.int32)),
    )(hs2d, gate_w)


def _ffn_body(meta_ref, xs_ref, wg_ref, wu_ref, wd_ref, out_ref):
    s = pl.program_id(0)
    lo = meta_ref[s, 2]
    hi = meta_ref[s, 3]
    x = xs_ref[...].astype(jnp.bfloat16)
    wg = wg_ref[0].astype(jnp.bfloat16)
    wu = wu_ref[0].astype(jnp.bfloat16)
    wd = wd_ref[0].astype(jnp.bfloat16)
    h = lax.dot_general(x, wg, (((1,), (1,)), ((), ())),
                        preferred_element_type=jnp.float32)
    u = lax.dot_general(x, wu, (((1,), (1,)), ((), ())),
                        preferred_element_type=jnp.float32)
    g = 0.5 * h * (1.0 + lax.erf(h * _SQRT_HALF))
    y = lax.dot_general((g * u).astype(jnp.bfloat16), wd,
                        (((1,), (1,)), ((), ())),
                        preferred_element_type=jnp.float32)
    row = lax.broadcasted_iota(jnp.int32, (TM, 1), 0)
    mask = (row >= lo) & (row < hi)
    out_ref[...] = jnp.where(mask, y, out_ref[...])


def _grouped_ffn(meta, xs, Wg, Wu, Wd):
    grid_spec = pltpu.PrefetchScalarGridSpec(
        num_scalar_prefetch=1,
        grid=(NS,),
        in_specs=[
            pl.BlockSpec((TM, D), lambda s, m: (m[s, 1], 0)),
            pl.BlockSpec((1, FF, D), lambda s, m: (m[s, 0], 0, 0)),
            pl.BlockSpec((1, FF, D), lambda s, m: (m[s, 0], 0, 0)),
            pl.BlockSpec((1, D, FF), lambda s, m: (m[s, 0], 0, 0)),
        ],
        out_specs=pl.BlockSpec((TM, D), lambda s, m: (m[s, 1], 0)),
    )
    return pl.pallas_call(
        _ffn_body,
        grid_spec=grid_spec,
        out_shape=jax.ShapeDtypeStruct((TK, D), jnp.float32),
        compiler_params=pltpu.CompilerParams(
            dimension_semantics=("arbitrary",)),
    )(meta, xs, Wg, Wu, Wd)


_NW = 32
_CPW = T // _NW


@functools.cache
def _sc_kernels():
    mesh = plsc.VectorSubcoreMesh(core_axis_name="c", subcore_axis_name="s")

    @functools.partial(
        pl.kernel, mesh=mesh,
        out_type=jax.ShapeDtypeStruct((TK, D), jnp.float32),
        scratch_types=[
            pltpu.VMEM((_CPW,), jnp.int32),
            pltpu.VMEM((_CPW,), jnp.int32),
            pltpu.VMEM((_CPW, D), jnp.float32),
            pltpu.SemaphoreType.DMA,
            pltpu.SemaphoreType.DMA,
        ],
    )
    def sc_scatter(hs_hbm, d0_hbm, d1_hbm, out_hbm,
                   i0_v, i1_v, rows_v, s0, s1):
        wid = lax.axis_index("s") * 2 + lax.axis_index("c")
        base = wid * _CPW
        pltpu.sync_copy(d0_hbm.at[pl.ds(base, _CPW)], i0_v)
        pltpu.sync_copy(d1_hbm.at[pl.ds(base, _CPW)], i1_v)
        pltpu.sync_copy(hs_hbm.at[pl.ds(base, _CPW)], rows_v)
        c0 = pltpu.async_copy(rows_v, out_hbm.at[i0_v], s0)
        c1 = pltpu.async_copy(rows_v, out_hbm.at[i1_v], s1)
        c0.wait()
        c1.wait()

    @functools.partial(
        pl.kernel, mesh=mesh,
        out_type=jax.ShapeDtypeStruct((T, D), jnp.float32),
        scratch_types=[
            pltpu.VMEM((_CPW,), jnp.int32),
            pltpu.VMEM((_CPW,), jnp.int32),
            pltpu.VMEM((_CPW, 16), jnp.float32),
            pltpu.VMEM((_CPW, 16), jnp.float32),
            pltpu.VMEM((_CPW, D), jnp.float32),
            pltpu.VMEM((_CPW, D), jnp.float32),
            pltpu.SemaphoreType.DMA,
            pltpu.SemaphoreType.DMA,
        ],
    )
    def sc_combine(ysw_hbm, pos0_hbm, pos1_hbm, w0_hbm, w1_hbm, out_hbm,
                   i0_v, i1_v, w0_v, w1_v, r0_v, r1_v, s0, s1):
        wid = lax.axis_index("s") * 2 + lax.axis_index("c")
        base = wid * _CPW
        pltpu.sync_copy(pos0_hbm.at[pl.ds(base, _CPW)], i0_v)
        pltpu.sync_copy(pos1_hbm.at[pl.ds(base, _CPW)], i1_v)
        pltpu.sync_copy(w0_hbm.at[pl.ds(base, _CPW)], w0_v)
        pltpu.sync_copy(w1_hbm.at[pl.ds(base, _CPW)], w1_v)
        c0 = pltpu.async_copy(ysw_hbm.at[i0_v], r0_v, s0)
        c1 = pltpu.async_copy(ysw_hbm.at[i1_v], r1_v, s1)
        c0.wait()
        c1.wait()

        def row_add(j, carry):
            a = w0_v[j, pl.ds(0, 16)]
            b = w1_v[j, pl.ds(0, 16)]
            for c in range(D // 16):
                sl = pl.ds(c * 16, 16)
                r0_v[j, sl] = r0_v[j, sl] * a + r1_v[j, sl] * b
            return carry

        lax.fori_loop(0, _CPW, row_add, 0)
        pltpu.sync_copy(r0_v, out_hbm.at[pl.ds(base, _CPW)])

    return sc_scatter, sc_combine


def kernel(hidden_states, gate_w, Wg, Wu, Wd):
    b, s, d = hidden_states.shape
    hs2d = hidden_states.reshape(T, D)
    w0, w1, dest0, dest1, meta = _router(hs2d, gate_w)
    d0 = dest0.reshape(T)
    d1 = dest1.reshape(T)
    sc_scatter, sc_combine = _sc_kernels()
    xs = sc_scatter(hs2d, d0, d1)
    ysw = _grouped_ffn(meta, xs, Wg, Wu, Wd)
    final = sc_combine(ysw, d0, d1, w0, w1)
    return final.reshape(b, s, d)
